# baseline (speedup 1.0000x reference)
import jax
import jax.numpy as jnp
from jax.experimental import pallas as pl
from jax.experimental.pallas import tpu as pltpu

KTOP = 31
RBLK = 512


def _xw_body(x_ref, w1_ref, out_ref):
    out_ref[...] = jax.lax.dot_general(
        x_ref[...], w1_ref[...], (((1,), (1,)), ((), ())),
        preferred_element_type=jnp.float32)


def _gcn1_body(adj_ref, xw_ref, b1_ref, w2_ref, out_ref):
    t = jnp.dot(adj_ref[...], xw_ref[...], preferred_element_type=jnp.float32)
    h1 = jnp.maximum(t + b1_ref[...], 0.0)
    out_ref[...] = jax.lax.dot_general(h1, w2_ref[...], (((1,), (1,)), ((), ())),
                                       preferred_element_type=jnp.float32)


def _gcn2_body(adj_ref, h1w_ref, b2_ref, out_ref):
    t = jnp.dot(adj_ref[...], h1w_ref[...], preferred_element_type=jnp.float32)
    t = t + b2_ref[...]
    nrm = jnp.sqrt(jnp.sum(t * t, axis=1, keepdims=True))
    nrm = jnp.maximum(nrm, 1e-12)
    out_ref[...] = t / nrm


CHUNK = 128


def _f2i(x):
    i = jax.lax.bitcast_convert_type(x, jnp.int32)
    return jnp.where(i < 0, i ^ jnp.int32(0x7FFFFFFF), i)


def _i2f(u):
    return jax.lax.bitcast_convert_type(
        jnp.where(u < 0, u ^ jnp.int32(0x7FFFFFFF), u), jnp.float32)


def _topk_mask(s_ref):
    rows, cols = s_ref.shape
    nchunk = cols // CHUNK
    kf = jnp.float32(KTOP)

    s0 = s_ref[...]
    lo = _f2i(jnp.min(s0, axis=1, keepdims=True))
    hi = _f2i(jnp.max(s0, axis=1, keepdims=True)) + 1
    cnt_lo = jnp.full((rows, 1), float(s_ref.shape[1]), dtype=jnp.float32)

    def cond(c):
        lo, hi, cnt_lo = c
        return jnp.max(hi - lo) > 1

    def body(c):
        lo, hi, cnt_lo = c
        mid = lo + jax.lax.shift_right_arithmetic(hi - lo, 1)
        cnt = jnp.sum((s_ref[...] >= _i2f(mid)).astype(jnp.float32), axis=1,
                      keepdims=True)
        ge = cnt >= kf
        return (jnp.where(ge, mid, lo), jnp.where(ge, hi, mid),
                jnp.where(ge, cnt, cnt_lo))

    lo, hi, cnt_ge = jax.lax.while_loop(cond, body, (lo, hi, cnt_lo))
    kth = _i2f(lo)

    s = s_ref[...]
    gt = s > kth
    eq = s == kth

    eqf = eq.astype(jnp.float32)
    e3 = eqf.reshape(rows * nchunk, CHUNK)
    tri = (jax.lax.broadcasted_iota(jnp.int32, (CHUNK, CHUNK), 0)
           < jax.lax.broadcasted_iota(jnp.int32, (CHUNK, CHUNK), 1)
           ).astype(jnp.float32)
    pin = jnp.dot(e3, tri, preferred_element_type=jnp.float32)
    csum = jnp.sum(e3, axis=1).reshape(rows, nchunk)
    cnt_eq = jnp.sum(csum, axis=1, keepdims=True)
    need = kf - (cnt_ge - cnt_eq)
    tri_c = (jax.lax.broadcasted_iota(jnp.int32, (nchunk, nchunk), 0)
             < jax.lax.broadcasted_iota(jnp.int32, (nchunk, nchunk), 1)
             ).astype(jnp.float32)
    carry = jnp.dot(csum, tri_c, preferred_element_type=jnp.float32)
    prefix = (pin.reshape(rows, nchunk, CHUNK)
              + carry.reshape(rows, nchunk, 1)).reshape(rows, cols)
    return gt | (eq & (prefix < need))


def _topk_body(hblk_ref, hall_ref, out_ref, s_ref):
    s_ref[...] = jax.lax.dot_general(
        hblk_ref[...], hall_ref[...], (((1,), (1,)), ((), ())),
        preferred_element_type=jnp.float32)
    mask = _topk_mask(s_ref)
    s = s_ref[...]
    out_ref[...] = jnp.where(mask & (s > 0.0), s, 0.0)


def kernel(x, adj, W1, b1, W2, b2):
    n, d = x.shape
    grid = n // RBLK
    fseq = dict(dimension_semantics=("arbitrary",))

    xw = pl.pallas_call(
        _xw_body,
        grid=(1,),
        in_specs=[
            pl.BlockSpec((n, d), lambda i: (0, 0)),
            pl.BlockSpec((d, d), lambda i: (0, 0)),
        ],
        out_specs=pl.BlockSpec((n, d), lambda i: (0, 0)),
        out_shape=jax.ShapeDtypeStruct((n, d), jnp.float32),
        compiler_params=pltpu.CompilerParams(**fseq),
    )(x, W1)

    h1w = pl.pallas_call(
        _gcn1_body,
        grid=(grid,),
        in_specs=[
            pl.BlockSpec((RBLK, n), lambda i: (i, 0)),
            pl.BlockSpec((n, d), lambda i: (0, 0)),
            pl.BlockSpec((1, d), lambda i: (0, 0)),
            pl.BlockSpec((d, d), lambda i: (0, 0)),
        ],
        out_specs=pl.BlockSpec((RBLK, d), lambda i: (i, 0)),
        out_shape=jax.ShapeDtypeStruct((n, d), jnp.float32),
        compiler_params=pltpu.CompilerParams(**fseq),
    )(adj, xw, b1.reshape(1, d), W2)

    h = pl.pallas_call(
        _gcn2_body,
        grid=(grid,),
        in_specs=[
            pl.BlockSpec((RBLK, n), lambda i: (i, 0)),
            pl.BlockSpec((n, d), lambda i: (0, 0)),
            pl.BlockSpec((1, d), lambda i: (0, 0)),
        ],
        out_specs=pl.BlockSpec((RBLK, d), lambda i: (i, 0)),
        out_shape=jax.ShapeDtypeStruct((n, d), jnp.float32),
        compiler_params=pltpu.CompilerParams(**fseq),
    )(adj, h1w, b2.reshape(1, d))

    out = pl.pallas_call(
        _topk_body,
        grid=(grid,),
        in_specs=[
            pl.BlockSpec((RBLK, d), lambda i: (i, 0)),
            pl.BlockSpec((n, d), lambda i: (0, 0)),
        ],
        out_specs=pl.BlockSpec((RBLK, n), lambda i: (i, 0)),
        out_shape=jax.ShapeDtypeStruct((n, n), jnp.float32),
        scratch_shapes=[pltpu.VMEM((RBLK, n), jnp.float32)],
        compiler_params=pltpu.CompilerParams(**fseq),
    )(h, h)

    return out


# speedup vs baseline: 1.0508x; 1.0508x over previous
import jax
import jax.numpy as jnp
from jax.experimental import pallas as pl
from jax.experimental.pallas import tpu as pltpu

KTOP = 31
RBLK = 512


def _xw_body(x_ref, w1_ref, out_ref):
    out_ref[...] = jax.lax.dot_general(
        x_ref[...], w1_ref[...], (((1,), (1,)), ((), ())),
        preferred_element_type=jnp.float32)


def _gcn1_body(adj_ref, xw_ref, b1_ref, w2_ref, out_ref):
    t = jnp.dot(adj_ref[...], xw_ref[...], preferred_element_type=jnp.float32)
    h1 = jnp.maximum(t + b1_ref[...], 0.0)
    out_ref[...] = jax.lax.dot_general(h1, w2_ref[...], (((1,), (1,)), ((), ())),
                                       preferred_element_type=jnp.float32)


def _gcn2_body(adj_ref, h1w_ref, b2_ref, out_ref):
    t = jnp.dot(adj_ref[...], h1w_ref[...], preferred_element_type=jnp.float32)
    t = t + b2_ref[...]
    nrm = jnp.sqrt(jnp.sum(t * t, axis=1, keepdims=True))
    nrm = jnp.maximum(nrm, 1e-12)
    out_ref[...] = t / nrm


CHUNK = 128


def _f2i(x):
    i = jax.lax.bitcast_convert_type(x, jnp.int32)
    return jnp.where(i < 0, i ^ jnp.int32(0x7FFFFFFF), i)


def _i2f(u):
    return jax.lax.bitcast_convert_type(
        jnp.where(u < 0, u ^ jnp.int32(0x7FFFFFFF), u), jnp.float32)


def _topk_mask(s_ref):
    rows, cols = s_ref.shape
    nchunk = cols // CHUNK
    kf = jnp.float32(KTOP)

    s0 = s_ref[...]
    lo = _f2i(jnp.min(s0, axis=1, keepdims=True))
    hi = _f2i(jnp.max(s0, axis=1, keepdims=True)) + 1
    cnt_lo = jnp.full((rows, 1), float(s_ref.shape[1]), dtype=jnp.float32)

    def cond(c):
        lo, hi, cnt_lo = c
        return jnp.max(hi - lo) > 1

    def body(c):
        lo, hi, cnt_lo = c
        mid = lo + jax.lax.shift_right_arithmetic(hi - lo, 1)
        cnt = jnp.sum((s_ref[...] >= _i2f(mid)).astype(jnp.float32), axis=1,
                      keepdims=True)
        ge = cnt >= kf
        return (jnp.where(ge, mid, lo), jnp.where(ge, hi, mid),
                jnp.where(ge, cnt, cnt_lo))

    lo, hi, cnt_ge = jax.lax.while_loop(cond, body, (lo, hi, cnt_lo))
    kth = _i2f(lo)

    s = s_ref[...]
    gt = s > kth
    eq = s == kth

    eqf = eq.astype(jnp.float32)
    tri = (jax.lax.broadcasted_iota(jnp.int32, (CHUNK, CHUNK), 0)
           < jax.lax.broadcasted_iota(jnp.int32, (CHUNK, CHUNK), 1)
           ).astype(jnp.float32)
    pins = []
    csums = []
    for c in range(nchunk):
        ec = eqf[:, c * CHUNK:(c + 1) * CHUNK]
        pins.append(jnp.dot(ec, tri, preferred_element_type=jnp.float32))
        csums.append(jnp.sum(ec, axis=1, keepdims=True))
    csum = jnp.concatenate(csums, axis=1)
    cnt_eq = jnp.sum(csum, axis=1, keepdims=True)
    need = kf - (cnt_ge - cnt_eq)
    tri_c = (jax.lax.broadcasted_iota(jnp.int32, (nchunk, nchunk), 0)
             < jax.lax.broadcasted_iota(jnp.int32, (nchunk, nchunk), 1)
             ).astype(jnp.float32)
    carry = jnp.dot(csum, tri_c, preferred_element_type=jnp.float32)
    prefix = jnp.concatenate(
        [pins[c] + carry[:, c:c + 1] for c in range(nchunk)], axis=1)
    return gt | (eq & (prefix < need))


def _topk_body(hblk_ref, hall_ref, out_ref, s_ref):
    s_ref[...] = jax.lax.dot_general(
        hblk_ref[...], hall_ref[...], (((1,), (1,)), ((), ())),
        preferred_element_type=jnp.float32)
    mask = _topk_mask(s_ref)
    s = s_ref[...]
    out_ref[...] = jnp.where(mask & (s > 0.0), s, 0.0)


def kernel(x, adj, W1, b1, W2, b2):
    n, d = x.shape
    grid = n // RBLK
    fseq = dict(dimension_semantics=("arbitrary",))

    xw = pl.pallas_call(
        _xw_body,
        grid=(1,),
        in_specs=[
            pl.BlockSpec((n, d), lambda i: (0, 0)),
            pl.BlockSpec((d, d), lambda i: (0, 0)),
        ],
        out_specs=pl.BlockSpec((n, d), lambda i: (0, 0)),
        out_shape=jax.ShapeDtypeStruct((n, d), jnp.float32),
        compiler_params=pltpu.CompilerParams(**fseq),
    )(x, W1)

    h1w = pl.pallas_call(
        _gcn1_body,
        grid=(grid,),
        in_specs=[
            pl.BlockSpec((RBLK, n), lambda i: (i, 0)),
            pl.BlockSpec((n, d), lambda i: (0, 0)),
            pl.BlockSpec((1, d), lambda i: (0, 0)),
            pl.BlockSpec((d, d), lambda i: (0, 0)),
        ],
        out_specs=pl.BlockSpec((RBLK, d), lambda i: (i, 0)),
        out_shape=jax.ShapeDtypeStruct((n, d), jnp.float32),
        compiler_params=pltpu.CompilerParams(**fseq),
    )(adj, xw, b1.reshape(1, d), W2)

    h = pl.pallas_call(
        _gcn2_body,
        grid=(grid,),
        in_specs=[
            pl.BlockSpec((RBLK, n), lambda i: (i, 0)),
            pl.BlockSpec((n, d), lambda i: (0, 0)),
            pl.BlockSpec((1, d), lambda i: (0, 0)),
        ],
        out_specs=pl.BlockSpec((RBLK, d), lambda i: (i, 0)),
        out_shape=jax.ShapeDtypeStruct((n, d), jnp.float32),
        compiler_params=pltpu.CompilerParams(**fseq),
    )(adj, h1w, b2.reshape(1, d))

    out = pl.pallas_call(
        _topk_body,
        grid=(grid,),
        in_specs=[
            pl.BlockSpec((RBLK, d), lambda i: (i, 0)),
            pl.BlockSpec((n, d), lambda i: (0, 0)),
        ],
        out_specs=pl.BlockSpec((RBLK, n), lambda i: (i, 0)),
        out_shape=jax.ShapeDtypeStruct((n, n), jnp.float32),
        scratch_shapes=[pltpu.VMEM((RBLK, n), jnp.float32)],
        compiler_params=pltpu.CompilerParams(**fseq),
    )(h, h)

    return out
